# Initial kernel scaffold; baseline (speedup 1.0000x reference)
import jax
import jax.numpy as jnp
from jax.experimental import pallas as pl


def kernel(x, edge_index, batch, Wq0, bq0, Wk0, bk0, Wv0, bv0, Ws0, bs0, Wq1, bq1, Wk1, bk1, Wv1, bv1, Ws1, bs1, Wq2, bq2, Wk2, bk2, Wv2, bv2, Ws2, bs2, fc_W, fc_b):
    raise NotImplementedError("write your pallas kernel here")


# speedup vs baseline: 1.0272x; 1.0272x over previous
import functools

import jax
import jax.numpy as jnp
from jax.experimental import pallas as pl
from jax.experimental.pallas import tpu as pltpu

N = 10000
E = 160000
HEADS = 8
C = 64
G = 128

ROW_BLK = 400
COL_BLK = 512


def _matmul_body(x_ref, w_ref, b_ref, o_ref):
    o_ref[...] = (
        jnp.dot(x_ref[...], w_ref[...], preferred_element_type=jnp.float32)
        + b_ref[...]
    )


def _fused_matmul(h, Wcat, bcat):
    n, din = h.shape
    dout = Wcat.shape[1]
    grid = (n // ROW_BLK, dout // COL_BLK)
    return pl.pallas_call(
        _matmul_body,
        grid=grid,
        in_specs=[
            pl.BlockSpec((ROW_BLK, din), lambda i, j: (i, 0)),
            pl.BlockSpec((din, COL_BLK), lambda i, j: (0, j)),
            pl.BlockSpec((1, COL_BLK), lambda i, j: (0, j)),
        ],
        out_specs=pl.BlockSpec((ROW_BLK, COL_BLK), lambda i, j: (i, j)),
        out_shape=jax.ShapeDtypeStruct((n, dout), jnp.float32),
    )(h, Wcat, bcat)


def _edge_stage(q, k, v, src, dst, concat):
    n = q.shape[0]
    qh = q.reshape(n, HEADS, C)
    kh = k.reshape(n, HEADS, C)
    vh = v.reshape(n, HEADS, C)
    logits = jnp.sum(qh[dst] * kh[src], axis=-1) / jnp.sqrt(float(C))
    e = jnp.exp(logits)
    s = jax.ops.segment_sum(e, dst, num_segments=n)
    a = e / (s[dst] + 1e-16)
    agg = jax.ops.segment_sum(vh[src] * a[:, :, None], dst, num_segments=n)
    if concat:
        return agg.reshape(n, HEADS * C)
    return agg.mean(axis=1)


def kernel(x, edge_index, batch, Wq0, bq0, Wk0, bk0, Wv0, bv0, Ws0, bs0,
           Wq1, bq1, Wk1, bk1, Wv1, bv1, Ws1, bs1,
           Wq2, bq2, Wk2, bk2, Wv2, bv2, Ws2, bs2, fc_W, fc_b):
    src = edge_index[0]
    dst = edge_index[1]

    def layer(h, Wq, bq, Wk, bk, Wv, bv, Ws, bs, concat):
        dout_s = Ws.shape[1]
        pad = 512 - dout_s if dout_s < 512 else 0
        Wsp = jnp.pad(Ws, ((0, 0), (0, pad))) if pad else Ws
        bsp = jnp.pad(bs, ((0, pad),)) if pad else bs
        Wcat = jnp.concatenate([Wq, Wk, Wv, Wsp], axis=1)
        bcat = jnp.concatenate([bq, bk, bv, bsp])[None, :]
        qkvs = _fused_matmul(h, Wcat, bcat)
        q = qkvs[:, 0:512]
        k = qkvs[:, 512:1024]
        v = qkvs[:, 1024:1536]
        skip = qkvs[:, 1536:1536 + dout_s]
        agg = _edge_stage(q, k, v, src, dst, concat)
        return agg + skip

    h = jax.nn.relu(layer(x, Wq0, bq0, Wk0, bk0, Wv0, bv0, Ws0, bs0, True))
    h = jax.nn.relu(layer(h, Wq1, bq1, Wk1, bk1, Wv1, bv1, Ws1, bs1, True))
    h = layer(h, Wq2, bq2, Wk2, bk2, Wv2, bv2, Ws2, bs2, False)

    sums = jax.ops.segment_sum(h, batch, num_segments=G)
    cnt = jax.ops.segment_sum(jnp.ones((h.shape[0],), h.dtype), batch,
                              num_segments=G)
    pooled = sums / jnp.maximum(cnt, 1.0)[:, None]
    out = pooled @ fc_W + fc_b
    return out.squeeze(-1)


# speedup vs baseline: 11.0150x; 10.7228x over previous
import functools

import jax
import jax.numpy as jnp
from jax import lax
from jax.experimental import pallas as pl
from jax.experimental.pallas import tpu as pltpu
from jax.experimental.pallas import tpu_sc as plsc

N = 10000
E = 160000
HEADS = 8
C = 64
G = 128

NP = 10240
NW = 32
NPT = NP // NW
CHN = 32
NCHUNK = NPT // CHN
EB = 64
EPAD = 8

ROW_BLK = 512

_LANES = 16


def _mm_body(x_ref, w_ref, b_ref, o_ref):
    o_ref[...] = (
        jnp.dot(x_ref[...], w_ref[...], preferred_element_type=jnp.float32)
        + b_ref[...]
    )


def _mm_add_body(x_ref, w_ref, b_ref, a_ref, o_ref):
    o_ref[...] = (
        jnp.dot(x_ref[...], w_ref[...], preferred_element_type=jnp.float32)
        + b_ref[...] + a_ref[...]
    )


def _mm(h, W, b, add=None):
    n, din = h.shape
    dout = W.shape[1]
    col_blk = min(512, dout)
    grid = (n // ROW_BLK, dout // col_blk)
    b2 = b[None, :]
    in_specs = [
        pl.BlockSpec((ROW_BLK, din), lambda i, j: (i, 0)),
        pl.BlockSpec((din, col_blk), lambda i, j: (0, j)),
        pl.BlockSpec((1, col_blk), lambda i, j: (0, j)),
    ]
    args = [h, W, b2]
    body = _mm_body
    if add is not None:
        in_specs.append(pl.BlockSpec((ROW_BLK, col_blk), lambda i, j: (i, j)))
        args.append(add)
        body = _mm_add_body
    return pl.pallas_call(
        body,
        grid=grid,
        in_specs=in_specs,
        out_specs=pl.BlockSpec((ROW_BLK, col_blk), lambda i, j: (i, j)),
        out_shape=jax.ShapeDtypeStruct((n, dout), jnp.float32),
    )(*args)


def _exp16(x):
    y = x * 1.4426950408889634
    n = (y + 12582912.0) - 12582912.0
    r = y - n
    p = 0.00015403530393381609
    p = p * r + 0.0013333558146428443
    p = p * r + 0.009618129107628477
    p = p * r + 0.05550410866482158
    p = p * r + 0.24022650695910072
    p = p * r + 0.6931471805599453
    p = p * r + 1.0
    ni = n.astype(jnp.int32)
    scale = plsc.bitcast(lax.shift_left(ni + 127, 23), jnp.float32)
    return p * scale


def _shuf(x, idx):
    return jnp.take_along_axis(x, idx, axis=0)


def _extract(vec, idx, iota):
    return jnp.max(jnp.where(iota == idx, vec, jnp.zeros_like(vec)))


def _make_edge_kernel(fuse_skip):
    QW = 1024 if fuse_skip else 512
    mesh = plsc.VectorSubcoreMesh(core_axis_name="c", subcore_axis_name="s", num_cores=2, num_subcores=16)

    @functools.partial(
        pl.kernel,
        out_type=jax.ShapeDtypeStruct((NP, 512), jnp.float32),
        mesh=mesh,
        compiler_params=pltpu.CompilerParams(needs_layout_passes=False),
        scratch_types=[
            pltpu.VMEM((48,), jnp.int32),
            pltpu.VMEM((EB + EPAD,), jnp.int32),
            pltpu.VMEM((EB + EPAD,), jnp.int32),
            pltpu.VMEM((CHN, QW), jnp.float32),
            pltpu.VMEM((EB + EPAD, 1024), jnp.float32),
            pltpu.VMEM((CHN, 512), jnp.float32),
            pltpu.VMEM((CHN, 16), jnp.float32),
            pltpu.SemaphoreType.DMA,
        ],
    )
    def ek(qs_hbm, kv_hbm, srcp_hbm, dstp_hbm, rowptr_hbm, out_hbm,
           rp, sidx, dw, qbuf, kvbuf, aggbuf, ssbuf, sem):
        wid = lax.axis_index("s") * 2 + lax.axis_index("c")
        nlo = wid * NPT
        iota = lax.iota(jnp.int32, _LANES)
        zero16 = jnp.zeros((_LANES,), jnp.float32)
        fold_idx = lax.bitwise_and(iota + 8, 15)

        def chunk_body(ch, _):
            n0 = nlo + ch * CHN
            pltpu.sync_copy(
                rowptr_hbm.at[pl.ds(pl.multiple_of(n0, 8), 48)], rp)
            pltpu.sync_copy(qs_hbm.at[pl.ds(n0, CHN)], qbuf)

            def zrow(dj, carry):
                for j in range(32):
                    aggbuf[dj, pl.ds(j * 16, 16)] = zero16
                ssbuf[dj] = zero16
                return carry
            lax.fori_loop(0, CHN, zrow, 0)

            rv0 = rp[pl.ds(0, 16)]
            rv1 = rp[pl.ds(32, 16)]
            e0 = _extract(rv0, 0, iota)
            e1 = _extract(rv1, 0, iota)
            nb = lax.shift_right_logical(e1 - e0 + (EB - 1), 6)

            def batch_body(b, carry):
                eb0 = e0 + b * EB
                ab = lax.bitwise_and(eb0, -8)
                off = eb0 - ab
                ab = pl.multiple_of(ab, 8)
                pltpu.sync_copy(srcp_hbm.at[pl.ds(ab, EB + EPAD)], sidx)
                pltpu.sync_copy(dstp_hbm.at[pl.ds(ab, EB + EPAD)], dw)
                pltpu.async_copy(kv_hbm.at[sidx], kvbuf, sem).wait()
                ecnt = jnp.minimum(e1 - eb0, EB)

                def edge_body(i, c2):
                    el = off + i
                    dsp = plsc.load_gather(
                        dw, [jnp.full((_LANES,), el, jnp.int32)])
                    dlv = dsp - n0
                    acc = zero16
                    for j in range(32):
                        qj = plsc.load_gather(qbuf, [dlv, iota + j * 16])
                        acc = acc + qj * kvbuf[el, pl.ds(j * 16, 16)]
                    acc = acc + _shuf(acc, fold_idx)
                    a = _exp16(acc)
                    plsc.addupdate_scatter(ssbuf, [dlv, iota], a)
                    for j in range(32):
                        plsc.addupdate_scatter(
                            aggbuf, [dlv, iota + j * 16],
                            a * kvbuf[el, pl.ds(512 + j * 16, 16)])
                    return c2
                lax.fori_loop(0, ecnt, edge_body, 0)
                return carry
            lax.fori_loop(0, nb, batch_body, 0)

            def fin(dj, carry):
                inv = 1.0 / (ssbuf[dj] + 1e-16)
                for j in range(32):
                    val = aggbuf[dj, pl.ds(j * 16, 16)] * inv
                    if fuse_skip:
                        val = jnp.maximum(
                            val + qbuf[dj, pl.ds(512 + j * 16, 16)], 0.0)
                    aggbuf[dj, pl.ds(j * 16, 16)] = val
                return carry
            lax.fori_loop(0, CHN, fin, 0)
            pltpu.sync_copy(aggbuf, out_hbm.at[pl.ds(n0, CHN)])
            return _
        lax.fori_loop(0, NCHUNK, chunk_body, 0)

    return ek


_edge_fused = _make_edge_kernel(True)
_edge_plain = _make_edge_kernel(False)


def _make_pool_kernel():
    mesh = plsc.VectorSubcoreMesh(core_axis_name="c", subcore_axis_name="s", num_cores=2, num_subcores=16)

    @functools.partial(
        pl.kernel,
        out_type=jax.ShapeDtypeStruct((NW, 16), jnp.float32),
        mesh=mesh,
        compiler_params=pltpu.CompilerParams(needs_layout_passes=False),
        scratch_types=[
            pltpu.VMEM((24,), jnp.int32),
            pltpu.VMEM((64, 64), jnp.float32),
            pltpu.VMEM((64,), jnp.float32),
            pltpu.VMEM((16,), jnp.float32),
            pltpu.VMEM((16,), jnp.float32),
            pltpu.SemaphoreType.DMA,
        ],
    )
    def pk(h2_hbm, gptr_hbm, fcw_hbm, fcb_hbm, out_hbm,
           gp, hbuf, fcw, fcb, res, sem):
        wid = lax.axis_index("s") * 2 + lax.axis_index("c")
        g0 = wid * 4
        base8 = lax.bitwise_and(g0, -8)
        o = g0 - base8
        iota = lax.iota(jnp.int32, _LANES)
        zero16 = jnp.zeros((_LANES,), jnp.float32)
        pltpu.sync_copy(
            gptr_hbm.at[pl.ds(pl.multiple_of(base8, 8), 24)], gp)
        pltpu.sync_copy(fcw_hbm, fcw)
        pltpu.sync_copy(fcb_hbm, fcb)
        gpv = gp[pl.ds(0, 16)]
        fcb_splat = _shuf(fcb[pl.ds(0, 16)], jnp.zeros((16,), jnp.int32))
        result = zero16
        for gl in range(4):
            gs = _extract(gpv, o + gl, iota)
            ge = _extract(gpv, o + gl + 1, iota)
            cnt = ge - gs
            a0 = lax.bitwise_and(gs, -8)
            nchk = lax.shift_right_logical(ge - a0 + 63, 6)

            def cb(t, sums):
                r0 = pl.multiple_of(a0 + t * 64, 8)
                pltpu.sync_copy(h2_hbm.at[pl.ds(r0, 64)], hbuf)
                lo = gs - r0
                hi = ge - r0

                def rb(r, sv):
                    m = jnp.logical_and(r >= lo, r < hi)
                    return tuple(
                        sv[j] + jnp.where(m, hbuf[r, pl.ds(j * 16, 16)],
                                          zero16)
                        for j in range(4))
                return lax.fori_loop(0, 64, rb, sums)
            sums = lax.fori_loop(0, nchk, cb,
                                 (zero16, zero16, zero16, zero16))
            cntv = jnp.maximum(jnp.full((_LANES,), cnt, jnp.int32), 1)
            inv = 1.0 / cntv.astype(jnp.float32)
            dot = zero16
            for j in range(4):
                dot = dot + sums[j] * inv * fcw[pl.ds(j * 16, 16)]
            for sh in (8, 4, 2, 1):
                dot = dot + _shuf(dot, lax.bitwise_xor(iota, sh))
            result = result + jnp.where(iota == gl, dot + fcb_splat, zero16)
        res[...] = result
        pltpu.sync_copy(res, out_hbm.at[wid])

    return pk


_pool = _make_pool_kernel()

import numpy as _np

_PERMC = (_np.arange(512) % 8) * 64 + _np.arange(512) // 8
_MMEAN = ((_np.arange(512)[:, None] // 8) ==
          _np.arange(64)[None, :]).astype(_np.float32) / 8.0


def kernel(x, edge_index, batch, Wq0, bq0, Wk0, bk0, Wv0, bv0, Ws0, bs0,
           Wq1, bq1, Wk1, bk1, Wv1, bv1, Ws1, bs1,
           Wq2, bq2, Wk2, bk2, Wv2, bv2, Ws2, bs2, fc_W, fc_b):
    src = edge_index[0]
    dst = edge_index[1]
    perm_e = jnp.argsort(dst)
    dst_s = dst[perm_e]
    src_s = src[perm_e]
    rowptr = jnp.searchsorted(dst_s, jnp.arange(NP + 48)).astype(jnp.int32)
    srcp = jnp.pad(src_s, (0, 128))
    dstp = jnp.pad(dst_s, (0, 128))

    x_p = jnp.pad(x, ((0, NP - N), (0, 0)))
    pc = _PERMC
    scale = 1.0 / jnp.sqrt(float(C))

    def layer01(h, Wq, bq, Wk, bk, Wv, bv, Ws, bs, permute_rows):
        if permute_rows:
            Wq, Wk, Wv, Ws = (W[pc, :] for W in (Wq, Wk, Wv, Ws))
        qs = _mm(h, jnp.concatenate([Wq[:, pc] * scale, Ws[:, pc]], axis=1),
                 jnp.concatenate([bq[pc] * scale, bs[pc]]))
        kv = _mm(h, jnp.concatenate([Wk[:, pc], Wv[:, pc]], axis=1),
                 jnp.concatenate([bk[pc], bv[pc]]))
        return _edge_fused(qs, kv, srcp, dstp, rowptr)

    h1 = layer01(x_p, Wq0, bq0, Wk0, bk0, Wv0, bv0, Ws0, bs0, False)
    h2in = layer01(h1, Wq1, bq1, Wk1, bk1, Wv1, bv1, Ws1, bs1, True)

    q2 = _mm(h2in, Wq2[pc, :][:, pc] * scale, bq2[pc] * scale)
    kv2 = _mm(h2in, jnp.concatenate(
        [Wk2[pc, :][:, pc], Wv2[pc, :][:, pc]], axis=1),
        jnp.concatenate([bk2[pc], bv2[pc]]))
    agg2 = _edge_plain(q2, kv2, srcp, dstp, rowptr)
    skip2 = _mm(h2in, Ws2[pc, :], bs2)
    h2 = _mm(agg2, jnp.asarray(_MMEAN), jnp.zeros((64,), jnp.float32),
             add=skip2)

    gptr = jnp.pad(
        jnp.searchsorted(batch, jnp.arange(G + 1)).astype(jnp.int32),
        (0, 15), constant_values=N)
    out2d = _pool(h2, gptr, fc_W[:, 0], jnp.pad(fc_b, (0, 15)))
    return out2d[:, :4].reshape(G)


# speedup vs baseline: 12.7762x; 1.1599x over previous
import functools

import jax
import jax.numpy as jnp
from jax import lax
from jax.experimental import pallas as pl
from jax.experimental.pallas import tpu as pltpu
from jax.experimental.pallas import tpu_sc as plsc

N = 10000
E = 160000
HEADS = 8
C = 64
G = 128

NP = 10240
NW = 32
NPT = NP // NW
CHN = 16
NCHUNK = NPT // CHN
EB = 32
EPAD = 8

ROW_BLK = 512

_LANES = 16


def _mm_body(x_ref, w_ref, b_ref, o_ref):
    o_ref[...] = (
        jnp.dot(x_ref[...], w_ref[...], preferred_element_type=jnp.float32)
        + b_ref[...]
    )


def _mm_add_body(x_ref, w_ref, b_ref, a_ref, o_ref):
    o_ref[...] = (
        jnp.dot(x_ref[...], w_ref[...], preferred_element_type=jnp.float32)
        + b_ref[...] + a_ref[...]
    )


def _mm(h, W, b, add=None):
    n, din = h.shape
    dout = W.shape[1]
    col_blk = min(512, dout)
    grid = (n // ROW_BLK, dout // col_blk)
    b2 = b[None, :]
    in_specs = [
        pl.BlockSpec((ROW_BLK, din), lambda i, j: (i, 0)),
        pl.BlockSpec((din, col_blk), lambda i, j: (0, j)),
        pl.BlockSpec((1, col_blk), lambda i, j: (0, j)),
    ]
    args = [h, W, b2]
    body = _mm_body
    if add is not None:
        in_specs.append(pl.BlockSpec((ROW_BLK, col_blk), lambda i, j: (i, j)))
        args.append(add)
        body = _mm_add_body
    return pl.pallas_call(
        body,
        grid=grid,
        in_specs=in_specs,
        out_specs=pl.BlockSpec((ROW_BLK, col_blk), lambda i, j: (i, j)),
        out_shape=jax.ShapeDtypeStruct((n, dout), jnp.float32),
    )(*args)


def _exp16(x):
    y = x * 1.4426950408889634
    n = (y + 12582912.0) - 12582912.0
    r = y - n
    p = 0.00015403530393381609
    p = p * r + 0.0013333558146428443
    p = p * r + 0.009618129107628477
    p = p * r + 0.05550410866482158
    p = p * r + 0.24022650695910072
    p = p * r + 0.6931471805599453
    p = p * r + 1.0
    ni = n.astype(jnp.int32)
    scale = plsc.bitcast(lax.shift_left(ni + 127, 23), jnp.float32)
    return p * scale


def _shuf(x, idx):
    return jnp.take_along_axis(x, idx, axis=0)


def _extract(vec, idx, iota):
    return jnp.max(jnp.where(iota == idx, vec, jnp.zeros_like(vec)))


def _make_edge_kernel(fuse_skip):
    QW = 1024 if fuse_skip else 512
    mesh = plsc.VectorSubcoreMesh(core_axis_name="c", subcore_axis_name="s", num_cores=2, num_subcores=16)

    NR = EB + EPAD

    @functools.partial(
        pl.kernel,
        out_type=jax.ShapeDtypeStruct((NP, 512), jnp.float32),
        mesh=mesh,
        compiler_params=pltpu.CompilerParams(needs_layout_passes=False),
        scratch_types=[
            pltpu.VMEM((48,), jnp.int32),
            pltpu.VMEM((2, NR), jnp.int32),
            pltpu.VMEM((2, NR), jnp.int32),
            pltpu.VMEM((CHN, QW), jnp.float32),
            pltpu.VMEM((2, NR, 1024), jnp.float32),
            pltpu.VMEM((CHN, 512), jnp.float32),
            pltpu.VMEM((CHN, 16), jnp.float32),
            pltpu.SemaphoreType.DMA,
            pltpu.SemaphoreType.DMA,
        ],
    )
    def ek(qs_hbm, kv_hbm, srcp_hbm, dstp_hbm, rowptr_hbm, out_hbm,
           rp, sidx, dw, qbuf, kvbuf, aggbuf, ssbuf, sem0, sem1):
        sems = (sem0, sem1)
        wid = lax.axis_index("s") * 2 + lax.axis_index("c")
        nlo = wid * NPT
        iota = lax.iota(jnp.int32, _LANES)
        zero16 = jnp.zeros((_LANES,), jnp.float32)
        fold_idx = lax.bitwise_and(iota + 8, 15)

        def chunk_body(ch, _):
            n0 = nlo + ch * CHN
            pltpu.sync_copy(
                rowptr_hbm.at[pl.ds(pl.multiple_of(n0, 8), 48)], rp)
            pltpu.sync_copy(qs_hbm.at[pl.ds(n0, CHN)], qbuf)

            def zrow(dj, carry):
                for j in range(32):
                    aggbuf[dj, pl.ds(j * 16, 16)] = zero16
                ssbuf[dj] = zero16
                return carry
            lax.fori_loop(0, CHN, zrow, 0)

            rv0 = rp[pl.ds(0, 16)]
            rv1 = rp[pl.ds(16, 16)]
            rv2 = rp[pl.ds(32, 16)]
            e0 = _extract(rv0, 0, iota)
            e1 = _extract((rv0, rv1, rv2)[CHN >> 4], CHN & 15, iota)
            nb = lax.shift_right_logical(e1 - e0 + (EB - 1), 5)

            def rpext(dlo):
                v = jnp.where(dlo < 16, rv0, jnp.where(dlo < 32, rv1, rv2))
                return _extract(v, lax.bitwise_and(dlo, 15), iota)

            def stage_and_fire(b, par):
                eb0 = e0 + b * EB
                ab = pl.multiple_of(lax.bitwise_and(eb0, -8), 8)
                pltpu.sync_copy(srcp_hbm.at[pl.ds(ab, NR)], sidx.at[par])
                pltpu.sync_copy(dstp_hbm.at[pl.ds(ab, NR)], dw.at[par])
                pltpu.async_copy(kv_hbm.at[sidx.at[par]], kvbuf.at[par],
                                 sems[par])

            @pl.when(nb > 0)
            def _prologue():
                stage_and_fire(0, 0)

            def process(b, par, ecur):
                eb0 = e0 + b * EB
                off = eb0 - lax.bitwise_and(eb0, -8)
                eend = jnp.minimum(e1, eb0 + EB)
                ecnt = eend - eb0
                d_first = _extract(plsc.load_gather(
                    dw, [jnp.full((_LANES,), par, jnp.int32),
                         jnp.full((_LANES,), off, jnp.int32)]), 0, iota)
                lend = jnp.maximum(off + ecnt - 1, 0)
                d_last = _extract(plsc.load_gather(
                    dw, [jnp.full((_LANES,), par, jnp.int32),
                         jnp.full((_LANES,), lend, jnp.int32)]), 0, iota)
                nnodes = jnp.where(ecnt > 0, d_last - d_first + 1, 0)
                nnodes = jnp.clip(nnodes, 0, CHN + 1)

                def node_body(t, ec2):
                    d = d_first + t
                    dlo = d - n0
                    ee = jnp.minimum(rpext(dlo + 1), eend)
                    qregs = [qbuf[dlo, pl.ds(j * 16, 16)] for j in range(32)]

                    def ebody(i, ssum):
                        el = off + (ec2 - eb0) + i
                        a0 = zero16
                        a1 = zero16
                        a2 = zero16
                        a3 = zero16
                        for j in range(0, 32, 4):
                            a0 = a0 + qregs[j] * kvbuf[
                                par, el, pl.ds(j * 16, 16)]
                            a1 = a1 + qregs[j + 1] * kvbuf[
                                par, el, pl.ds(j * 16 + 16, 16)]
                            a2 = a2 + qregs[j + 2] * kvbuf[
                                par, el, pl.ds(j * 16 + 32, 16)]
                            a3 = a3 + qregs[j + 3] * kvbuf[
                                par, el, pl.ds(j * 16 + 48, 16)]
                        acc = (a0 + a1) + (a2 + a3)
                        acc = acc + _shuf(acc, fold_idx)
                        a = _exp16(acc)
                        for j in range(32):
                            plsc.addupdate(
                                aggbuf.at[dlo, pl.ds(j * 16, 16)],
                                a * kvbuf[par, el, pl.ds(512 + j * 16, 16)])
                        return ssum + a
                    ssum = lax.fori_loop(0, ee - ec2, ebody, zero16)
                    plsc.addupdate(ssbuf.at[dlo], ssum)
                    return jnp.maximum(ee, ec2)
                return lax.fori_loop(0, nnodes, node_body, ecur)

            npair = lax.shift_right_logical(nb + 1, 1)

            def pair_body(g, ecur):
                for par in (0, 1):
                    b = 2 * g + par

                    @pl.when(b + 1 < nb)
                    def _prefetch():
                        stage_and_fire(b + 1, 1 - par)

                    @pl.when(b < nb)
                    def _waitcur():
                        pltpu.make_async_copy(
                            kv_hbm.at[sidx.at[par]], kvbuf.at[par],
                            sems[par]).wait()
                    ecur = process(b, par, ecur)
                return ecur
            lax.fori_loop(0, npair, pair_body, e0)

            def fin(dj, carry):
                inv = 1.0 / (ssbuf[dj] + 1e-16)
                for j in range(32):
                    val = aggbuf[dj, pl.ds(j * 16, 16)] * inv
                    if fuse_skip:
                        val = jnp.maximum(
                            val + qbuf[dj, pl.ds(512 + j * 16, 16)], 0.0)
                    aggbuf[dj, pl.ds(j * 16, 16)] = val
                return carry
            lax.fori_loop(0, CHN, fin, 0)
            pltpu.sync_copy(aggbuf, out_hbm.at[pl.ds(n0, CHN)])
            return _
        lax.fori_loop(0, NCHUNK, chunk_body, 0)

    return ek


_edge_fused = _make_edge_kernel(True)
_edge_plain = _make_edge_kernel(False)


def _make_pool_kernel():
    mesh = plsc.VectorSubcoreMesh(core_axis_name="c", subcore_axis_name="s", num_cores=2, num_subcores=16)

    @functools.partial(
        pl.kernel,
        out_type=jax.ShapeDtypeStruct((NW, 16), jnp.float32),
        mesh=mesh,
        compiler_params=pltpu.CompilerParams(needs_layout_passes=False),
        scratch_types=[
            pltpu.VMEM((24,), jnp.int32),
            pltpu.VMEM((64, 64), jnp.float32),
            pltpu.VMEM((64,), jnp.float32),
            pltpu.VMEM((16,), jnp.float32),
            pltpu.VMEM((16,), jnp.float32),
            pltpu.SemaphoreType.DMA,
        ],
    )
    def pk(h2_hbm, gptr_hbm, fcw_hbm, fcb_hbm, out_hbm,
           gp, hbuf, fcw, fcb, res, sem):
        wid = lax.axis_index("s") * 2 + lax.axis_index("c")
        g0 = wid * 4
        base8 = lax.bitwise_and(g0, -8)
        o = g0 - base8
        iota = lax.iota(jnp.int32, _LANES)
        zero16 = jnp.zeros((_LANES,), jnp.float32)
        pltpu.sync_copy(
            gptr_hbm.at[pl.ds(pl.multiple_of(base8, 8), 24)], gp)
        pltpu.sync_copy(fcw_hbm, fcw)
        pltpu.sync_copy(fcb_hbm, fcb)
        gpv = gp[pl.ds(0, 16)]
        fcb_splat = _shuf(fcb[pl.ds(0, 16)], jnp.zeros((16,), jnp.int32))
        result = zero16
        for gl in range(4):
            gs = _extract(gpv, o + gl, iota)
            ge = _extract(gpv, o + gl + 1, iota)
            cnt = ge - gs
            a0 = lax.bitwise_and(gs, -8)
            nchk = lax.shift_right_logical(ge - a0 + 63, 6)

            def cb(t, sums):
                r0 = pl.multiple_of(a0 + t * 64, 8)
                pltpu.sync_copy(h2_hbm.at[pl.ds(r0, 64)], hbuf)
                lo = gs - r0
                hi = ge - r0

                def rb(r, sv):
                    m = jnp.logical_and(r >= lo, r < hi)
                    return tuple(
                        sv[j] + jnp.where(m, hbuf[r, pl.ds(j * 16, 16)],
                                          zero16)
                        for j in range(4))
                return lax.fori_loop(0, 64, rb, sums)
            sums = lax.fori_loop(0, nchk, cb,
                                 (zero16, zero16, zero16, zero16))
            cntv = jnp.maximum(jnp.full((_LANES,), cnt, jnp.int32), 1)
            inv = 1.0 / cntv.astype(jnp.float32)
            dot = zero16
            for j in range(4):
                dot = dot + sums[j] * inv * fcw[pl.ds(j * 16, 16)]
            for sh in (8, 4, 2, 1):
                dot = dot + _shuf(dot, lax.bitwise_xor(iota, sh))
            result = result + jnp.where(iota == gl, dot + fcb_splat, zero16)
        res[...] = result
        pltpu.sync_copy(res, out_hbm.at[wid])

    return pk


_pool = _make_pool_kernel()

import numpy as _np

_PERMC = (_np.arange(512) % 8) * 64 + _np.arange(512) // 8
_MMEAN = ((_np.arange(512)[:, None] // 8) ==
          _np.arange(64)[None, :]).astype(_np.float32) / 8.0


def kernel(x, edge_index, batch, Wq0, bq0, Wk0, bk0, Wv0, bv0, Ws0, bs0,
           Wq1, bq1, Wk1, bk1, Wv1, bv1, Ws1, bs1,
           Wq2, bq2, Wk2, bk2, Wv2, bv2, Ws2, bs2, fc_W, fc_b):
    src = edge_index[0]
    dst = edge_index[1]
    perm_e = jnp.argsort(dst)
    dst_s = dst[perm_e]
    src_s = src[perm_e]
    rowptr = jnp.searchsorted(dst_s, jnp.arange(NP + 48)).astype(jnp.int32)
    srcp = jnp.pad(src_s, (0, 128))
    dstp = jnp.pad(dst_s, (0, 128))

    x_p = jnp.pad(x, ((0, NP - N), (0, 0)))
    pc = _PERMC
    scale = 1.0 / jnp.sqrt(float(C))

    def layer01(h, Wq, bq, Wk, bk, Wv, bv, Ws, bs, permute_rows):
        if permute_rows:
            Wq, Wk, Wv, Ws = (W[pc, :] for W in (Wq, Wk, Wv, Ws))
        qs = _mm(h, jnp.concatenate([Wq[:, pc] * scale, Ws[:, pc]], axis=1),
                 jnp.concatenate([bq[pc] * scale, bs[pc]]))
        kv = _mm(h, jnp.concatenate([Wk[:, pc], Wv[:, pc]], axis=1),
                 jnp.concatenate([bk[pc], bv[pc]]))
        return _edge_fused(qs, kv, srcp, dstp, rowptr)

    h1 = layer01(x_p, Wq0, bq0, Wk0, bk0, Wv0, bv0, Ws0, bs0, False)
    h2in = layer01(h1, Wq1, bq1, Wk1, bk1, Wv1, bv1, Ws1, bs1, True)

    q2 = _mm(h2in, Wq2[pc, :][:, pc] * scale, bq2[pc] * scale)
    kv2 = _mm(h2in, jnp.concatenate(
        [Wk2[pc, :][:, pc], Wv2[pc, :][:, pc]], axis=1),
        jnp.concatenate([bk2[pc], bv2[pc]]))
    agg2 = _edge_plain(q2, kv2, srcp, dstp, rowptr)
    skip2 = _mm(h2in, Ws2[pc, :], bs2)
    h2 = _mm(agg2, jnp.asarray(_MMEAN), jnp.zeros((64,), jnp.float32),
             add=skip2)

    gptr = jnp.pad(
        jnp.searchsorted(batch, jnp.arange(G + 1)).astype(jnp.int32),
        (0, 15), constant_values=N)
    out2d = _pool(h2, gptr, fc_W[:, 0], jnp.pad(fc_b, (0, 15)))
    return out2d[:, :4].reshape(G)


# speedup vs baseline: 20.2968x; 1.5886x over previous
import functools

import jax
import jax.numpy as jnp
from jax import lax
from jax.experimental import pallas as pl
from jax.experimental.pallas import tpu as pltpu
from jax.experimental.pallas import tpu_sc as plsc

N = 10000
E = 160000
HEADS = 8
C = 64
G = 128

NP = 10240
NW = 32
NPT = NP // NW
CHN = 16
NCHUNK = NPT // CHN
EB = 32
EPAD = 8

ROW_BLK = 512

_LANES = 16


def _mm_body(x_ref, w_ref, b_ref, o_ref):
    o_ref[...] = (
        jnp.dot(x_ref[...], w_ref[...], preferred_element_type=jnp.float32)
        + b_ref[...]
    )


def _mm_add_body(x_ref, w_ref, b_ref, a_ref, o_ref):
    o_ref[...] = (
        jnp.dot(x_ref[...], w_ref[...], preferred_element_type=jnp.float32)
        + b_ref[...] + a_ref[...]
    )


def _mm(h, W, b, add=None):
    n, din = h.shape
    dout = W.shape[1]
    col_blk = min(512, dout)
    grid = (n // ROW_BLK, dout // col_blk)
    b2 = b[None, :]
    in_specs = [
        pl.BlockSpec((ROW_BLK, din), lambda i, j: (i, 0)),
        pl.BlockSpec((din, col_blk), lambda i, j: (0, j)),
        pl.BlockSpec((1, col_blk), lambda i, j: (0, j)),
    ]
    args = [h, W, b2]
    body = _mm_body
    if add is not None:
        in_specs.append(pl.BlockSpec((ROW_BLK, col_blk), lambda i, j: (i, j)))
        args.append(add)
        body = _mm_add_body
    return pl.pallas_call(
        body,
        grid=grid,
        in_specs=in_specs,
        out_specs=pl.BlockSpec((ROW_BLK, col_blk), lambda i, j: (i, j)),
        out_shape=jax.ShapeDtypeStruct((n, dout), jnp.float32),
    )(*args)


def _exp16(x):
    y = x * 1.4426950408889634
    n = (y + 12582912.0) - 12582912.0
    r = y - n
    p = 0.00015403530393381609
    p = p * r + 0.0013333558146428443
    p = p * r + 0.009618129107628477
    p = p * r + 0.05550410866482158
    p = p * r + 0.24022650695910072
    p = p * r + 0.6931471805599453
    p = p * r + 1.0
    ni = n.astype(jnp.int32)
    scale = plsc.bitcast(lax.shift_left(ni + 127, 23), jnp.float32)
    return p * scale


def _shuf(x, idx):
    return jnp.take_along_axis(x, idx, axis=0)


def _extract(vec, idx, iota):
    return jnp.max(jnp.where(iota == idx, vec, jnp.zeros_like(vec)))


def _make_edge_kernel(fuse_skip):
    QW = 1024 if fuse_skip else 512
    mesh = plsc.VectorSubcoreMesh(core_axis_name="c", subcore_axis_name="s", num_cores=2, num_subcores=16)

    NR = EB + EPAD

    @functools.partial(
        pl.kernel,
        out_type=jax.ShapeDtypeStruct((NP, 512), jnp.float32),
        mesh=mesh,
        compiler_params=pltpu.CompilerParams(needs_layout_passes=False),
        scratch_types=[
            pltpu.VMEM((48,), jnp.int32),
            pltpu.VMEM((2, NR), jnp.int32),
            pltpu.VMEM((2, NR), jnp.int32),
            pltpu.VMEM((CHN, QW), jnp.float32),
            pltpu.VMEM((2, NR, 1024), jnp.float32),
            pltpu.VMEM((CHN, 512), jnp.float32),
            pltpu.VMEM((CHN, 16), jnp.float32),
            pltpu.VMEM((NR, 16), jnp.float32),
            pltpu.SemaphoreType.DMA,
            pltpu.SemaphoreType.DMA,
        ],
    )
    def ek(qs_hbm, kv_hbm, srcp_hbm, dstp_hbm, rowptr_hbm, out_hbm,
           rp, sidx, dw, qbuf, kvbuf, aggbuf, ssbuf, abuf, sem0, sem1):
        sems = (sem0, sem1)
        wid = lax.axis_index("s") * 2 + lax.axis_index("c")
        nlo = wid * NPT
        iota = lax.iota(jnp.int32, _LANES)
        zero16 = jnp.zeros((_LANES,), jnp.float32)
        fold_idx = lax.bitwise_and(iota + 8, 15)

        def chunk_body(ch, _):
            n0 = nlo + ch * CHN
            pltpu.sync_copy(
                rowptr_hbm.at[pl.ds(pl.multiple_of(n0, 8), 48)], rp)
            pltpu.sync_copy(qs_hbm.at[pl.ds(n0, CHN)], qbuf)

            def zrow(dj, carry):
                for j in range(32):
                    aggbuf[dj, pl.ds(j * 16, 16)] = zero16
                ssbuf[dj] = zero16
                return carry
            lax.fori_loop(0, CHN, zrow, 0)

            rv0 = rp[pl.ds(0, 16)]
            rv1 = rp[pl.ds(16, 16)]
            rv2 = rp[pl.ds(32, 16)]
            e0 = _extract(rv0, 0, iota)
            e1 = _extract((rv0, rv1, rv2)[CHN >> 4], CHN & 15, iota)
            nb = lax.shift_right_logical(e1 - e0 + (EB - 1), 5)

            def rpext(dlo):
                v = jnp.where(dlo < 16, rv0, jnp.where(dlo < 32, rv1, rv2))
                return _extract(v, lax.bitwise_and(dlo, 15), iota)

            def stage_and_fire(b, par):
                eb0 = e0 + b * EB
                ab = pl.multiple_of(lax.bitwise_and(eb0, -8), 8)
                pltpu.sync_copy(srcp_hbm.at[pl.ds(ab, NR)], sidx.at[par])
                pltpu.sync_copy(dstp_hbm.at[pl.ds(ab, NR)], dw.at[par])
                pltpu.async_copy(kv_hbm.at[sidx.at[par]], kvbuf.at[par],
                                 sems[par])

            @pl.when(nb > 0)
            def _prologue():
                stage_and_fire(0, 0)

            def process(b, par, ecur):
                eb0 = e0 + b * EB
                off = eb0 - lax.bitwise_and(eb0, -8)
                eend = jnp.minimum(e1, eb0 + EB)
                ecnt = eend - eb0
                d_first = _extract(plsc.load_gather(
                    dw, [jnp.full((_LANES,), par, jnp.int32),
                         jnp.full((_LANES,), off, jnp.int32)]), 0, iota)
                lend = jnp.maximum(off + ecnt - 1, 0)
                d_last = _extract(plsc.load_gather(
                    dw, [jnp.full((_LANES,), par, jnp.int32),
                         jnp.full((_LANES,), lend, jnp.int32)]), 0, iota)
                nnodes = jnp.where(ecnt > 0, d_last - d_first + 1, 0)
                nnodes = jnp.clip(nnodes, 0, CHN + 1)

                def node_body(t, ec2):
                    d = d_first + t
                    dlo = d - n0
                    ee = jnp.minimum(rpext(dlo + 1), eend)
                    base = off + (ec2 - eb0)
                    qregs = [qbuf[dlo, pl.ds(j * 16, 16)] for j in range(32)]

                    def ebodyA(i, ssum):
                        el = base + i
                        a0 = zero16
                        a1 = zero16
                        a2 = zero16
                        a3 = zero16
                        for j in range(0, 32, 4):
                            a0 = a0 + qregs[j] * kvbuf[
                                par, el, pl.ds(j * 16, 16)]
                            a1 = a1 + qregs[j + 1] * kvbuf[
                                par, el, pl.ds(j * 16 + 16, 16)]
                            a2 = a2 + qregs[j + 2] * kvbuf[
                                par, el, pl.ds(j * 16 + 32, 16)]
                            a3 = a3 + qregs[j + 3] * kvbuf[
                                par, el, pl.ds(j * 16 + 48, 16)]
                        acc = (a0 + a1) + (a2 + a3)
                        acc = acc + _shuf(acc, fold_idx)
                        a = _exp16(acc)
                        abuf[el] = a
                        return ssum + a
                    ssum = lax.fori_loop(0, ee - ec2, ebodyA, zero16)
                    plsc.addupdate(ssbuf.at[dlo], ssum)

                    def ebodyB(i, vr):
                        el = base + i
                        a = abuf[el]
                        return tuple(
                            vr[j] + a * kvbuf[par, el,
                                              pl.ds(512 + j * 16, 16)]
                            for j in range(32))
                    vr = lax.fori_loop(0, ee - ec2, ebodyB,
                                       (zero16,) * 32)
                    for j in range(32):
                        plsc.addupdate(aggbuf.at[dlo, pl.ds(j * 16, 16)],
                                       vr[j])
                    return jnp.maximum(ee, ec2)
                return lax.fori_loop(0, nnodes, node_body, ecur)

            npair = lax.shift_right_logical(nb + 1, 1)

            def pair_body(g, ecur):
                for par in (0, 1):
                    b = 2 * g + par

                    @pl.when(b + 1 < nb)
                    def _prefetch():
                        stage_and_fire(b + 1, 1 - par)

                    @pl.when(b < nb)
                    def _waitcur():
                        pltpu.make_async_copy(
                            kv_hbm.at[sidx.at[par]], kvbuf.at[par],
                            sems[par]).wait()
                    ecur = process(b, par, ecur)
                return ecur
            lax.fori_loop(0, npair, pair_body, e0)

            def fin(dj, carry):
                inv = 1.0 / (ssbuf[dj] + 1e-16)
                for j in range(32):
                    val = aggbuf[dj, pl.ds(j * 16, 16)] * inv
                    if fuse_skip:
                        val = jnp.maximum(
                            val + qbuf[dj, pl.ds(512 + j * 16, 16)], 0.0)
                    aggbuf[dj, pl.ds(j * 16, 16)] = val
                return carry
            lax.fori_loop(0, CHN, fin, 0)
            pltpu.sync_copy(aggbuf, out_hbm.at[pl.ds(n0, CHN)])
            return _
        lax.fori_loop(0, NCHUNK, chunk_body, 0)

    return ek


_edge_fused = _make_edge_kernel(True)
_edge_plain = _make_edge_kernel(False)


def _make_pool_kernel():
    mesh = plsc.VectorSubcoreMesh(core_axis_name="c", subcore_axis_name="s", num_cores=2, num_subcores=16)

    @functools.partial(
        pl.kernel,
        out_type=jax.ShapeDtypeStruct((NW, 16), jnp.float32),
        mesh=mesh,
        compiler_params=pltpu.CompilerParams(needs_layout_passes=False),
        scratch_types=[
            pltpu.VMEM((24,), jnp.int32),
            pltpu.VMEM((64, 64), jnp.float32),
            pltpu.VMEM((64,), jnp.float32),
            pltpu.VMEM((16,), jnp.float32),
            pltpu.VMEM((16,), jnp.float32),
            pltpu.SemaphoreType.DMA,
        ],
    )
    def pk(h2_hbm, gptr_hbm, fcw_hbm, fcb_hbm, out_hbm,
           gp, hbuf, fcw, fcb, res, sem):
        wid = lax.axis_index("s") * 2 + lax.axis_index("c")
        g0 = wid * 4
        base8 = lax.bitwise_and(g0, -8)
        o = g0 - base8
        iota = lax.iota(jnp.int32, _LANES)
        zero16 = jnp.zeros((_LANES,), jnp.float32)
        pltpu.sync_copy(
            gptr_hbm.at[pl.ds(pl.multiple_of(base8, 8), 24)], gp)
        pltpu.sync_copy(fcw_hbm, fcw)
        pltpu.sync_copy(fcb_hbm, fcb)
        gpv = gp[pl.ds(0, 16)]
        fcb_splat = _shuf(fcb[pl.ds(0, 16)], jnp.zeros((16,), jnp.int32))
        result = zero16
        for gl in range(4):
            gs = _extract(gpv, o + gl, iota)
            ge = _extract(gpv, o + gl + 1, iota)
            cnt = ge - gs
            a0 = lax.bitwise_and(gs, -8)
            nchk = lax.shift_right_logical(ge - a0 + 63, 6)

            def cb(t, sums):
                r0 = pl.multiple_of(a0 + t * 64, 8)
                pltpu.sync_copy(h2_hbm.at[pl.ds(r0, 64)], hbuf)
                lo = gs - r0
                hi = ge - r0

                def rb(r, sv):
                    m = jnp.logical_and(r >= lo, r < hi)
                    return tuple(
                        sv[j] + jnp.where(m, hbuf[r, pl.ds(j * 16, 16)],
                                          zero16)
                        for j in range(4))
                return lax.fori_loop(0, 64, rb, sums)
            sums = lax.fori_loop(0, nchk, cb,
                                 (zero16, zero16, zero16, zero16))
            cntv = jnp.maximum(jnp.full((_LANES,), cnt, jnp.int32), 1)
            inv = 1.0 / cntv.astype(jnp.float32)
            dot = zero16
            for j in range(4):
                dot = dot + sums[j] * inv * fcw[pl.ds(j * 16, 16)]
            for sh in (8, 4, 2, 1):
                dot = dot + _shuf(dot, lax.bitwise_xor(iota, sh))
            result = result + jnp.where(iota == gl, dot + fcb_splat, zero16)
        res[...] = result
        pltpu.sync_copy(res, out_hbm.at[wid])

    return pk


_pool = _make_pool_kernel()

import numpy as _np

_PERMC = (_np.arange(512) % 8) * 64 + _np.arange(512) // 8
_MMEAN = ((_np.arange(512)[:, None] // 8) ==
          _np.arange(64)[None, :]).astype(_np.float32) / 8.0


def kernel(x, edge_index, batch, Wq0, bq0, Wk0, bk0, Wv0, bv0, Ws0, bs0,
           Wq1, bq1, Wk1, bk1, Wv1, bv1, Ws1, bs1,
           Wq2, bq2, Wk2, bk2, Wv2, bv2, Ws2, bs2, fc_W, fc_b):
    src = edge_index[0]
    dst = edge_index[1]
    dst_s, src_s = jax.lax.sort((dst, src), num_keys=1)
    rowptr = jnp.searchsorted(dst_s, jnp.arange(NP + 48)).astype(jnp.int32)
    srcp = jnp.pad(src_s, (0, 128))
    dstp = jnp.pad(dst_s, (0, 128))

    x_p = jnp.pad(x, ((0, NP - N), (0, 0)))
    pc = _PERMC
    scale = 1.0 / jnp.sqrt(float(C))

    def layer01(h, Wq, bq, Wk, bk, Wv, bv, Ws, bs, permute_rows):
        if permute_rows:
            Wq, Wk, Wv, Ws = (W[pc, :] for W in (Wq, Wk, Wv, Ws))
        qs = _mm(h, jnp.concatenate([Wq[:, pc] * scale, Ws[:, pc]], axis=1),
                 jnp.concatenate([bq[pc] * scale, bs[pc]]))
        kv = _mm(h, jnp.concatenate([Wk[:, pc], Wv[:, pc]], axis=1),
                 jnp.concatenate([bk[pc], bv[pc]]))
        return _edge_fused(qs, kv, srcp, dstp, rowptr)

    h1 = layer01(x_p, Wq0, bq0, Wk0, bk0, Wv0, bv0, Ws0, bs0, False)
    h2in = layer01(h1, Wq1, bq1, Wk1, bk1, Wv1, bv1, Ws1, bs1, True)

    q2 = _mm(h2in, Wq2[pc, :][:, pc] * scale, bq2[pc] * scale)
    kv2 = _mm(h2in, jnp.concatenate(
        [Wk2[pc, :][:, pc], Wv2[pc, :][:, pc]], axis=1),
        jnp.concatenate([bk2[pc], bv2[pc]]))
    agg2 = _edge_plain(q2, kv2, srcp, dstp, rowptr)
    skip2 = _mm(h2in, Ws2[pc, :], bs2)
    h2 = _mm(agg2, jnp.asarray(_MMEAN), jnp.zeros((64,), jnp.float32),
             add=skip2)

    gptr = jnp.pad(
        jnp.searchsorted(batch, jnp.arange(G + 1)).astype(jnp.int32),
        (0, 15), constant_values=N)
    out2d = _pool(h2, gptr, fc_W[:, 0], jnp.pad(fc_b, (0, 15)))
    return out2d[:, :4].reshape(G)


# speedup vs baseline: 20.3539x; 1.0028x over previous
import functools

import jax
import jax.numpy as jnp
from jax import lax
from jax.experimental import pallas as pl
from jax.experimental.pallas import tpu as pltpu
from jax.experimental.pallas import tpu_sc as plsc

N = 10000
E = 160000
HEADS = 8
C = 64
G = 128

NP = 10240
NW = 32
NPT = NP // NW
CHN = 16
NCHUNK = NPT // CHN
EB = 32
EPAD = 8

ROW_BLK = 512

_LANES = 16


def _mm_body(x_ref, w_ref, b_ref, o_ref):
    o_ref[...] = (
        jnp.dot(x_ref[...], w_ref[...], preferred_element_type=jnp.float32)
        + b_ref[...]
    )


def _mm_add_body(x_ref, w_ref, b_ref, a_ref, o_ref):
    o_ref[...] = (
        jnp.dot(x_ref[...], w_ref[...], preferred_element_type=jnp.float32)
        + b_ref[...] + a_ref[...]
    )


def _mm(h, W, b, add=None):
    n, din = h.shape
    dout = W.shape[1]
    col_blk = min(512, dout)
    grid = (n // ROW_BLK, dout // col_blk)
    b2 = b[None, :]
    in_specs = [
        pl.BlockSpec((ROW_BLK, din), lambda i, j: (i, 0)),
        pl.BlockSpec((din, col_blk), lambda i, j: (0, j)),
        pl.BlockSpec((1, col_blk), lambda i, j: (0, j)),
    ]
    args = [h, W, b2]
    body = _mm_body
    if add is not None:
        in_specs.append(pl.BlockSpec((ROW_BLK, col_blk), lambda i, j: (i, j)))
        args.append(add)
        body = _mm_add_body
    return pl.pallas_call(
        body,
        grid=grid,
        in_specs=in_specs,
        out_specs=pl.BlockSpec((ROW_BLK, col_blk), lambda i, j: (i, j)),
        out_shape=jax.ShapeDtypeStruct((n, dout), jnp.float32),
    )(*args)


def _exp16(x):
    y = x * 1.4426950408889634
    n = (y + 12582912.0) - 12582912.0
    r = y - n
    p = 0.00015403530393381609
    p = p * r + 0.0013333558146428443
    p = p * r + 0.009618129107628477
    p = p * r + 0.05550410866482158
    p = p * r + 0.24022650695910072
    p = p * r + 0.6931471805599453
    p = p * r + 1.0
    ni = n.astype(jnp.int32)
    scale = plsc.bitcast(lax.shift_left(ni + 127, 23), jnp.float32)
    return p * scale


def _shuf(x, idx):
    return jnp.take_along_axis(x, idx, axis=0)


def _extract(vec, idx, iota):
    return jnp.max(jnp.where(iota == idx, vec, jnp.zeros_like(vec)))


def _make_edge_kernel(fuse_skip):
    QW = 1024 if fuse_skip else 512
    mesh = plsc.VectorSubcoreMesh(core_axis_name="c", subcore_axis_name="s", num_cores=2, num_subcores=16)

    NR = EB + EPAD

    @functools.partial(
        pl.kernel,
        out_type=jax.ShapeDtypeStruct((NP, 512), jnp.float32),
        mesh=mesh,
        compiler_params=pltpu.CompilerParams(needs_layout_passes=False),
        scratch_types=[
            pltpu.VMEM((48,), jnp.int32),
            pltpu.VMEM((2, NR), jnp.int32),
            pltpu.VMEM((2, NR), jnp.int32),
            pltpu.VMEM((CHN, QW), jnp.float32),
            pltpu.VMEM((2, NR, 1024), jnp.float32),
            pltpu.VMEM((CHN, 512), jnp.float32),
            pltpu.VMEM((CHN, 16), jnp.float32),
            pltpu.VMEM((NR, 16), jnp.float32),
            pltpu.SemaphoreType.DMA,
            pltpu.SemaphoreType.DMA,
        ],
    )
    def ek(qs_hbm, kv_hbm, srcp_hbm, dstp_hbm, rowptr_hbm, out_hbm,
           rp, sidx, dw, qbuf, kvbuf, aggbuf, ssbuf, abuf, sem0, sem1):
        sems = (sem0, sem1)
        wid = lax.axis_index("s") * 2 + lax.axis_index("c")
        nlo = wid * NPT
        iota = lax.iota(jnp.int32, _LANES)
        zero16 = jnp.zeros((_LANES,), jnp.float32)
        fold_idx = lax.bitwise_and(iota + 8, 15)

        def chunk_body(ch, _):
            n0 = nlo + ch * CHN
            pltpu.sync_copy(
                rowptr_hbm.at[pl.ds(pl.multiple_of(n0, 8), 48)], rp)
            pltpu.sync_copy(qs_hbm.at[pl.ds(n0, CHN)], qbuf)

            def zrow(dj, carry):
                for j in range(32):
                    aggbuf[dj, pl.ds(j * 16, 16)] = zero16
                ssbuf[dj] = zero16
                return carry
            lax.fori_loop(0, CHN, zrow, 0)

            rv0 = rp[pl.ds(0, 16)]
            rv1 = rp[pl.ds(16, 16)]
            rv2 = rp[pl.ds(32, 16)]
            e0 = _extract(rv0, 0, iota)
            e1 = _extract((rv0, rv1, rv2)[CHN >> 4], CHN & 15, iota)
            nb = lax.shift_right_logical(e1 - e0 + (EB - 1), 5)

            def rpext(dlo):
                v = jnp.where(dlo < 16, rv0, jnp.where(dlo < 32, rv1, rv2))
                return _extract(v, lax.bitwise_and(dlo, 15), iota)

            def stage_and_fire(b, par):
                eb0 = e0 + b * EB
                ab = pl.multiple_of(lax.bitwise_and(eb0, -8), 8)
                pltpu.sync_copy(srcp_hbm.at[pl.ds(ab, NR)], sidx.at[par])
                pltpu.sync_copy(dstp_hbm.at[pl.ds(ab, NR)], dw.at[par])
                pltpu.async_copy(kv_hbm.at[sidx.at[par]], kvbuf.at[par],
                                 sems[par])

            @pl.when(nb > 0)
            def _prologue():
                stage_and_fire(0, 0)

            def process(b, par, ecur):
                eb0 = e0 + b * EB
                off = eb0 - lax.bitwise_and(eb0, -8)
                eend = jnp.minimum(e1, eb0 + EB)
                ecnt = eend - eb0
                d_first = _extract(plsc.load_gather(
                    dw, [jnp.full((_LANES,), par, jnp.int32),
                         jnp.full((_LANES,), off, jnp.int32)]), 0, iota)
                lend = jnp.maximum(off + ecnt - 1, 0)
                d_last = _extract(plsc.load_gather(
                    dw, [jnp.full((_LANES,), par, jnp.int32),
                         jnp.full((_LANES,), lend, jnp.int32)]), 0, iota)
                nnodes = jnp.where(ecnt > 0, d_last - d_first + 1, 0)
                nnodes = jnp.clip(nnodes, 0, CHN + 1)

                def node_body(t, ec2):
                    d = d_first + t
                    dlo = d - n0
                    ee = jnp.minimum(rpext(dlo + 1), eend)
                    base = off + (ec2 - eb0)
                    qregs = [qbuf[dlo, pl.ds(j * 16, 16)] for j in range(32)]

                    def ebodyA(i, ssum):
                        el = base + i
                        a0 = zero16
                        a1 = zero16
                        a2 = zero16
                        a3 = zero16
                        for j in range(0, 32, 4):
                            a0 = a0 + qregs[j] * kvbuf[
                                par, el, pl.ds(j * 16, 16)]
                            a1 = a1 + qregs[j + 1] * kvbuf[
                                par, el, pl.ds(j * 16 + 16, 16)]
                            a2 = a2 + qregs[j + 2] * kvbuf[
                                par, el, pl.ds(j * 16 + 32, 16)]
                            a3 = a3 + qregs[j + 3] * kvbuf[
                                par, el, pl.ds(j * 16 + 48, 16)]
                        acc = (a0 + a1) + (a2 + a3)
                        acc = acc + _shuf(acc, fold_idx)
                        a = _exp16(acc)
                        abuf[el] = a
                        return ssum + a
                    ssum = lax.fori_loop(0, ee - ec2, ebodyA, zero16)
                    plsc.addupdate(ssbuf.at[dlo], ssum)

                    def ebodyB(i, vr):
                        el = base + i
                        a = abuf[el]
                        return tuple(
                            vr[j] + a * kvbuf[par, el,
                                              pl.ds(512 + j * 16, 16)]
                            for j in range(32))
                    vr = lax.fori_loop(0, ee - ec2, ebodyB,
                                       (zero16,) * 32)
                    for j in range(32):
                        plsc.addupdate(aggbuf.at[dlo, pl.ds(j * 16, 16)],
                                       vr[j])
                    return jnp.maximum(ee, ec2)
                return lax.fori_loop(0, nnodes, node_body, ecur)

            npair = lax.shift_right_logical(nb + 1, 1)

            def pair_body(g, ecur):
                for par in (0, 1):
                    b = 2 * g + par

                    @pl.when(b + 1 < nb)
                    def _prefetch():
                        stage_and_fire(b + 1, 1 - par)

                    @pl.when(b < nb)
                    def _waitcur():
                        pltpu.make_async_copy(
                            kv_hbm.at[sidx.at[par]], kvbuf.at[par],
                            sems[par]).wait()
                    ecur = process(b, par, ecur)
                return ecur
            lax.fori_loop(0, npair, pair_body, e0)

            def fin(dj, carry):
                inv = 1.0 / (ssbuf[dj] + 1e-16)
                for j in range(32):
                    val = aggbuf[dj, pl.ds(j * 16, 16)] * inv
                    if fuse_skip:
                        val = jnp.maximum(
                            val + qbuf[dj, pl.ds(512 + j * 16, 16)], 0.0)
                    aggbuf[dj, pl.ds(j * 16, 16)] = val
                return carry
            lax.fori_loop(0, CHN, fin, 0)
            pltpu.sync_copy(aggbuf, out_hbm.at[pl.ds(n0, CHN)])
            return _
        lax.fori_loop(0, NCHUNK, chunk_body, 0)

    return ek


_edge_fused = _make_edge_kernel(True)
_edge_plain = _make_edge_kernel(False)


def _make_pool_kernel():
    mesh = plsc.VectorSubcoreMesh(core_axis_name="c", subcore_axis_name="s", num_cores=2, num_subcores=16)

    @functools.partial(
        pl.kernel,
        out_type=jax.ShapeDtypeStruct((NW, 16), jnp.float32),
        mesh=mesh,
        compiler_params=pltpu.CompilerParams(needs_layout_passes=False),
        scratch_types=[
            pltpu.VMEM((24,), jnp.int32),
            pltpu.VMEM((64, 64), jnp.float32),
            pltpu.VMEM((64,), jnp.float32),
            pltpu.VMEM((16,), jnp.float32),
            pltpu.VMEM((16,), jnp.float32),
            pltpu.SemaphoreType.DMA,
        ],
    )
    def pk(h2_hbm, gptr_hbm, fcw_hbm, fcb_hbm, out_hbm,
           gp, hbuf, fcw, fcb, res, sem):
        wid = lax.axis_index("s") * 2 + lax.axis_index("c")
        g0 = wid * 4
        base8 = lax.bitwise_and(g0, -8)
        o = g0 - base8
        iota = lax.iota(jnp.int32, _LANES)
        zero16 = jnp.zeros((_LANES,), jnp.float32)
        pltpu.sync_copy(
            gptr_hbm.at[pl.ds(pl.multiple_of(base8, 8), 24)], gp)
        pltpu.sync_copy(fcw_hbm, fcw)
        pltpu.sync_copy(fcb_hbm, fcb)
        gpv = gp[pl.ds(0, 16)]
        fcb_splat = _shuf(fcb[pl.ds(0, 16)], jnp.zeros((16,), jnp.int32))
        result = zero16
        for gl in range(4):
            gs = _extract(gpv, o + gl, iota)
            ge = _extract(gpv, o + gl + 1, iota)
            cnt = ge - gs
            a0 = lax.bitwise_and(gs, -8)
            nchk = lax.shift_right_logical(ge - a0 + 63, 6)

            def cb(t, sums):
                r0 = pl.multiple_of(a0 + t * 64, 8)
                pltpu.sync_copy(h2_hbm.at[pl.ds(r0, 64)], hbuf)
                lo = gs - r0
                hi = ge - r0

                def rb(r, sv):
                    m = jnp.logical_and(r >= lo, r < hi)
                    return tuple(
                        sv[j] + jnp.where(m, hbuf[r, pl.ds(j * 16, 16)],
                                          zero16)
                        for j in range(4))
                return lax.fori_loop(0, 64, rb, sums)
            sums = lax.fori_loop(0, nchk, cb,
                                 (zero16, zero16, zero16, zero16))
            cntv = jnp.maximum(jnp.full((_LANES,), cnt, jnp.int32), 1)
            inv = 1.0 / cntv.astype(jnp.float32)
            dot = zero16
            for j in range(4):
                dot = dot + sums[j] * inv * fcw[pl.ds(j * 16, 16)]
            for sh in (8, 4, 2, 1):
                dot = dot + _shuf(dot, lax.bitwise_xor(iota, sh))
            result = result + jnp.where(iota == gl, dot + fcb_splat, zero16)
        res[...] = result
        pltpu.sync_copy(res, out_hbm.at[wid])

    return pk


_pool = _make_pool_kernel()

import numpy as _np

_PERMC = (_np.arange(512) % 8) * 64 + _np.arange(512) // 8
_MMEAN = ((_np.arange(512)[:, None] // 8) ==
          _np.arange(64)[None, :]).astype(_np.float32) / 8.0


def kernel(x, edge_index, batch, Wq0, bq0, Wk0, bk0, Wv0, bv0, Ws0, bs0,
           Wq1, bq1, Wk1, bk1, Wv1, bv1, Ws1, bs1,
           Wq2, bq2, Wk2, bk2, Wv2, bv2, Ws2, bs2, fc_W, fc_b):
    src = edge_index[0]
    dst = edge_index[1]
    comb = jax.lax.sort(dst * 16384 + src)
    src_s = jnp.bitwise_and(comb, 16383)
    dst_s = jnp.right_shift(comb, 14)
    rowptr = jnp.searchsorted(
        comb, jnp.arange(NP + 48) * 16384).astype(jnp.int32)
    srcp = jnp.pad(src_s, (0, 128))
    dstp = jnp.pad(dst_s, (0, 128))

    x_p = jnp.pad(x, ((0, NP - N), (0, 0)))
    pc = _PERMC
    scale = 1.0 / jnp.sqrt(float(C))

    def layer01(h, Wq, bq, Wk, bk, Wv, bv, Ws, bs, permute_rows):
        if permute_rows:
            Wq, Wk, Wv, Ws = (W[pc, :] for W in (Wq, Wk, Wv, Ws))
        qs = _mm(h, jnp.concatenate([Wq[:, pc] * scale, Ws[:, pc]], axis=1),
                 jnp.concatenate([bq[pc] * scale, bs[pc]]))
        kv = _mm(h, jnp.concatenate([Wk[:, pc], Wv[:, pc]], axis=1),
                 jnp.concatenate([bk[pc], bv[pc]]))
        return _edge_fused(qs, kv, srcp, dstp, rowptr)

    h1 = layer01(x_p, Wq0, bq0, Wk0, bk0, Wv0, bv0, Ws0, bs0, False)
    h2in = layer01(h1, Wq1, bq1, Wk1, bk1, Wv1, bv1, Ws1, bs1, True)

    q2 = _mm(h2in, Wq2[pc, :][:, pc] * scale, bq2[pc] * scale)
    kv2 = _mm(h2in, jnp.concatenate(
        [Wk2[pc, :][:, pc], Wv2[pc, :][:, pc]], axis=1),
        jnp.concatenate([bk2[pc], bv2[pc]]))
    agg2 = _edge_plain(q2, kv2, srcp, dstp, rowptr)
    skip2 = _mm(h2in, Ws2[pc, :], bs2)
    h2 = _mm(agg2, jnp.asarray(_MMEAN), jnp.zeros((64,), jnp.float32),
             add=skip2)

    gptr = jnp.pad(
        jnp.searchsorted(batch, jnp.arange(G + 1)).astype(jnp.int32),
        (0, 15), constant_values=N)
    out2d = _pool(h2, gptr, fc_W[:, 0], jnp.pad(fc_b, (0, 15)))
    return out2d[:, :4].reshape(G)
